# Initial kernel scaffold; baseline (speedup 1.0000x reference)
import jax
import jax.numpy as jnp
from jax.experimental import pallas as pl


def kernel(classifications, regressions, annotations):
    raise NotImplementedError("write your pallas kernel here")


# speedup vs baseline: 198.4629x; 198.4629x over previous
import jax
import jax.numpy as jnp
from jax.experimental import pallas as pl
from jax.experimental.pallas import tpu as pltpu


def _body(ct_ref, ann_ref, dec_ref, esc_ref, rs_ref, cls_ref, reg_ref):
    B = ct_ref.shape[0]
    N = float(ct_ref.shape[2] * ct_ref.shape[3])

    def focal(x, t):
        af = jnp.where(t, 0.25, 0.75)
        w = jnp.where(t, 1.0 - x, x)
        bce = jnp.where(t, -jnp.log(x), -jnp.log(1.0 - x))
        return af * w * w * bce

    def huber(r):
        d = jnp.abs(r)
        return jnp.where(d <= 1.0, 0.5 * d * d, d - 0.5)

    cls_acc = jnp.float32(0.0)
    reg_acc = jnp.float32(0.0)
    for j in range(B):
        x0 = jnp.clip(ct_ref[j, 0], 0.0001, 1.0 - 0.0001)
        x1 = jnp.clip(ct_ref[j, 1], 0.0001, 1.0 - 0.0001)
        m = ann_ref[j, 0] == 1
        l1 = ann_ref[j, 1] == 1
        t0 = jnp.logical_and(m, jnp.logical_not(l1))
        t1 = jnp.logical_and(m, l1)
        s = jnp.sum(focal(x0, t0)) + jnp.sum(focal(x1, t1))
        memb = (dec_ref[j] == 1).astype(jnp.float32)
        esc = esc_ref[...].astype(jnp.float32)
        p = (jnp.sum(jnp.max(memb * (1.0 - esc), axis=1))
             + jnp.sum(memb * esc))
        cls_acc = cls_acc + s / p
        reg_acc = reg_acc + (p * huber(rs_ref[j, 1])
                             + (N - p) * huber(rs_ref[j, 0])) / N
    cls_ref[0] = cls_acc / B
    reg_ref[0] = reg_acc / B


def kernel(classifications, regressions, annotations):
    B, N, C = classifications.shape
    L = annotations.shape[2]
    ndec = -(-L // 10)
    ct = classifications.transpose(0, 2, 1).reshape(B, C, 128, N // 128)
    annp = jnp.pad(annotations, ((0, 0), (0, 0), (0, N - L)))
    annp = annp.reshape(B, 2, 128, N // 128)
    dec = jnp.pad(annotations[:, 0, :], ((0, 0), (0, ndec * 10 - L)))
    dec = dec.reshape(B, ndec, 10)
    rs = regressions[:, :2, 0]
    i_arr = jnp.arange(L, dtype=jnp.int32)
    gt_ctr = ((i_arr - 4) + (i_arr + 4)) / 2.0
    anchor_ctr = (jnp.floor(gt_ctr / 10.0) * 10.0
                  + jnp.ceil(gt_ctr / 10.0 + 0.1) * 10.0) / 2.0
    m_i = anchor_ctr.astype(jnp.int32)
    esc = (m_i != (i_arr // 10) * 10 + 5).astype(jnp.int32)
    esc = jnp.pad(esc, (0, ndec * 10 - L)).reshape(ndec, 10)
    cls, reg = pl.pallas_call(
        _body,
        out_shape=(jax.ShapeDtypeStruct((1,), jnp.float32),
                   jax.ShapeDtypeStruct((1,), jnp.float32)),
        in_specs=[pl.BlockSpec(memory_space=pltpu.VMEM),
                  pl.BlockSpec(memory_space=pltpu.VMEM),
                  pl.BlockSpec(memory_space=pltpu.VMEM),
                  pl.BlockSpec(memory_space=pltpu.VMEM),
                  pl.BlockSpec(memory_space=pltpu.SMEM)],
        out_specs=(pl.BlockSpec(memory_space=pltpu.SMEM),
                   pl.BlockSpec(memory_space=pltpu.SMEM)),
    )(ct, annp, dec, esc, rs)
    return cls, reg
